# baseline (speedup 1.0000x reference)
import functools

import jax
import jax.numpy as jnp
from jax import lax
from jax.experimental import pallas as pl
from jax.experimental.pallas import tpu as pltpu
from jax.experimental.pallas import tpu_sc as plsc

T_SC = 2048
_NW = 32
_RPW = T_SC // _NW


def _tc_body(x_ref, xq_ref, idx_ref):
    xq_ref[...] = x_ref[...] * 2.0
    idx_ref[...] = jnp.zeros_like(idx_ref)


def _sc_body(x_hbm, out_hbm, buf):
    wid = lax.axis_index("s") * 2 + lax.axis_index("c")
    base = wid * _RPW
    pltpu.sync_copy(x_hbm.at[pl.ds(base, _RPW)], buf)
    pltpu.sync_copy(buf, out_hbm.at[pl.ds(base, _RPW)])


@jax.jit
def kernel(x, W_in, b_in, W_out, b_out):
    B, N, D = x.shape
    T = B * N
    T_TC = T - T_SC
    R = 1024
    x2 = x.reshape(T, D)

    mesh = plsc.VectorSubcoreMesh(core_axis_name="c", subcore_axis_name="s")
    sc_copy = functools.partial(
        pl.kernel,
        out_type=jax.ShapeDtypeStruct((T_SC, D), jnp.float32),
        mesh=mesh,
        scratch_types=[pltpu.VMEM((_RPW, D), jnp.float32)],
    )(_sc_body)

    xq_sc = sc_copy(x2[T_TC:])

    xq_tc, idx = pl.pallas_call(
        _tc_body,
        grid=(T_TC // R,),
        in_specs=[pl.BlockSpec((R, D), lambda i: (i, 0))],
        out_specs=[
            pl.BlockSpec((R, D), lambda i: (i, 0)),
            pl.BlockSpec((R, 1), lambda i: (i, 0)),
        ],
        out_shape=[
            jax.ShapeDtypeStruct((T_TC, D), jnp.float32),
            jax.ShapeDtypeStruct((T_TC, 1), jnp.int32),
        ],
    )(x2[:T_TC])

    xq = jnp.concatenate([xq_tc, xq_sc], axis=0)
    idx_full = jnp.concatenate([idx, jnp.zeros((T_SC, 1), jnp.int32)], axis=0)
    return (xq.reshape(B, N, D), idx_full.reshape(B, N),
            jnp.zeros((), jnp.float32))


# speedup vs baseline: 2.0260x; 2.0260x over previous
import functools

import jax
import jax.numpy as jnp
import numpy as np
from jax.experimental import pallas as pl
from jax.experimental.pallas import tpu as pltpu

_LEVELS = np.array([8, 8, 8, 5, 5, 5], dtype=np.int32)
_C = 128
_EPS = 1e-3

_lv = np.full((_C,), 3, dtype=np.float64)
_lv[: len(_LEVELS)] = _LEVELS
_half_l = (_lv - 1.0) * (1.0 - _EPS) / 2.0
_offset = np.where(_lv % 2 == 0, 0.5, 0.0)
_shift = np.arctanh(_offset / _half_l)
_half_width = np.floor(_lv / 2.0)
_basis = np.zeros((_C,), dtype=np.float64)
_basis[: len(_LEVELS)] = np.concatenate([[1], np.cumprod(_LEVELS[:-1])])

_CONSTS = np.zeros((8, _C), dtype=np.float32)
_CONSTS[0] = _half_l
_CONSTS[1] = _shift
_CONSTS[2] = _offset
_CONSTS[3] = _half_width
_CONSTS[4] = 1.0 / _half_width
_CONSTS[5] = _basis

_R = 1024


def _body(x_ref, win_ref, bin_ref, wout_ref, bout_ref, c_ref,
          xq_ref, idx_ref, codes_ref):
    p = pl.program_id(0)
    i = pl.program_id(1)
    ng = pl.num_programs(1)

    @pl.when(p == 0)
    def _phase_in():
        j = ng - 1 - i
        x = x_ref[...]
        z = jnp.dot(x, win_ref[...], preferred_element_type=jnp.float32,
                    precision=jax.lax.Precision.DEFAULT)
        z = z + bin_ref[...]
        half_l = c_ref[0:1, :]
        shift = c_ref[1:2, :]
        offset = c_ref[2:3, :]
        half_w = c_ref[3:4, :]
        inv_half_w = c_ref[4:5, :]
        basis = c_ref[5:6, :]
        bounded = jnp.tanh(z + shift) * half_l - offset
        q = jnp.round(bounded)
        codes = q * inv_half_w
        scaled = q + half_w
        idx_ref[...] = jnp.sum(scaled * basis, axis=-1,
                               keepdims=True).astype(jnp.int32)
        codes_ref[pl.ds(j * _R, _R), :] = codes

    @pl.when(p == 1)
    def _phase_out():
        codes = codes_ref[pl.ds(i * _R, _R), :]
        out = jnp.dot(codes, wout_ref[...], preferred_element_type=jnp.float32,
                      precision=jax.lax.Precision.DEFAULT)
        xq_ref[...] = out + bout_ref[...]


@jax.jit
def kernel(x, W_in, b_in, W_out, b_out):
    B, N, D = x.shape
    T = B * N
    cb = W_in.shape[1]
    G = T // _R

    x2 = x.reshape(T, D)
    win = jnp.zeros((D, _C), jnp.float32).at[:, :cb].set(W_in)
    bin_ = jnp.zeros((1, _C), jnp.float32).at[0, :cb].set(b_in)
    wout = jnp.zeros((_C, D), jnp.float32).at[:cb, :].set(W_out)
    bout = b_out.reshape(1, D)

    xq, idx = pl.pallas_call(
        _body,
        grid=(2, G),
        in_specs=[
            pl.BlockSpec((_R, D), lambda p, i, g=G: ((1 - p) * (g - 1 - i), 0)),
            pl.BlockSpec((D, _C), lambda p, i: (0, 0)),
            pl.BlockSpec((1, _C), lambda p, i: (0, 0)),
            pl.BlockSpec((_C, D), lambda p, i: (0, 0)),
            pl.BlockSpec((1, D), lambda p, i: (0, 0)),
            pl.BlockSpec((8, _C), lambda p, i: (0, 0)),
        ],
        out_specs=[
            pl.BlockSpec((_R, D), lambda p, i: (p * i, 0)),
            pl.BlockSpec((_R, 1), lambda p, i, g=G: ((1 - p) * (g - 1 - i), 0)),
        ],
        out_shape=[
            jax.ShapeDtypeStruct((T, D), jnp.float32),
            jax.ShapeDtypeStruct((T, 1), jnp.int32),
        ],
        scratch_shapes=[pltpu.VMEM((T, _C), jnp.float32)],
    )(x2, win, bin_, wout, bout, jnp.asarray(_CONSTS))

    commit_loss = jnp.zeros((), dtype=jnp.float32)
    return (xq.reshape(B, N, D), idx.reshape(B, N), commit_loss)
